# baseline (speedup 1.0000x reference)
import jax
import jax.numpy as jnp
from jax import lax
from jax.experimental import pallas as pl
from jax.experimental.pallas import tpu as pltpu

POS_DIM = 3
FEAT_DIM = 128
HIDDEN = 256
BLOCK_N = 4096


def _mlp_block_kernel(post_ref, w_ref, w1_ref, b1_ref, w2pt_ref, b2pt_ref,
                      w2w_ref, b2w_ref, out_post_ref, out_w_ref):
    w = w_ref[...]
    h = jnp.maximum(
        jnp.dot(w.astype(jnp.bfloat16), w1_ref[...],
                preferred_element_type=jnp.float32)
        + b1_ref[...], 0.0)
    hb = h.astype(jnp.bfloat16)
    dpt = lax.dot_general(w2pt_ref[...], hb, (((1,), (1,)), ((), ())),
                          preferred_element_type=jnp.float32)
    dw = jnp.dot(hb, w2w_ref[...], preferred_element_type=jnp.float32)
    out_post_ref[...] = post_ref[...] + dpt + b2pt_ref[...]
    out_w_ref[...] = w + dw + b2w_ref[...]


def kernel(positions, weights, batch, W1, b1, W2, b2):
    del batch
    n = weights.shape[0]
    grid = (n // BLOCK_N,)

    posT = positions.T
    W1 = W1.astype(jnp.bfloat16)
    W2pT = W2[:, :POS_DIM].T.astype(jnp.bfloat16)
    W2w = W2[:, POS_DIM:].astype(jnp.bfloat16)
    b1r = b1.reshape(1, HIDDEN)
    b2pT = b2[:POS_DIM].reshape(POS_DIM, 1)
    b2w = b2[POS_DIM:].reshape(1, FEAT_DIM)

    row_block = lambda i: (i, 0)
    col_block = lambda i: (0, i)
    rep = lambda i: (0, 0)
    out_posT, out_w = pl.pallas_call(
        _mlp_block_kernel,
        grid=grid,
        in_specs=[
            pl.BlockSpec((POS_DIM, BLOCK_N), col_block),
            pl.BlockSpec((BLOCK_N, FEAT_DIM), row_block),
            pl.BlockSpec((FEAT_DIM, HIDDEN), rep),
            pl.BlockSpec((1, HIDDEN), rep),
            pl.BlockSpec((POS_DIM, HIDDEN), rep),
            pl.BlockSpec((POS_DIM, 1), rep),
            pl.BlockSpec((HIDDEN, FEAT_DIM), rep),
            pl.BlockSpec((1, FEAT_DIM), rep),
        ],
        out_specs=[
            pl.BlockSpec((POS_DIM, BLOCK_N), col_block),
            pl.BlockSpec((BLOCK_N, FEAT_DIM), row_block),
        ],
        out_shape=[
            jax.ShapeDtypeStruct((POS_DIM, n), jnp.float32),
            jax.ShapeDtypeStruct((n, FEAT_DIM), jnp.float32),
        ],
        compiler_params=pltpu.CompilerParams(
            dimension_semantics=("parallel",),
        ),
    )(posT, weights, W1, b1r, W2pT, b2pT, W2w, b2w)
    return out_posT.T, out_w


# speedup vs baseline: 1.1106x; 1.1106x over previous
import jax
import jax.numpy as jnp
from jax import lax
from jax.experimental import pallas as pl
from jax.experimental.pallas import tpu as pltpu

POS_DIM = 3
FEAT_DIM = 128
HIDDEN = 256
BLOCK_N = 8192


def _mlp_block_kernel(post_ref, w_ref, w1_ref, b1_ref, w2pt_ref, b2pt_ref,
                      w2w_ref, b2w_ref, out_post_ref, out_w_ref):
    w = w_ref[...]
    h = jnp.maximum(
        jnp.dot(w, w1_ref[...], preferred_element_type=jnp.float32)
        + b1_ref[...], 0.0)
    hb = h
    dpt = lax.dot_general(w2pt_ref[...], hb, (((1,), (1,)), ((), ())),
                          preferred_element_type=jnp.float32)
    dw = jnp.dot(hb, w2w_ref[...], preferred_element_type=jnp.float32)
    out_post_ref[...] = post_ref[...] + dpt + b2pt_ref[...]
    out_w_ref[...] = w + dw + b2w_ref[...]


def kernel(positions, weights, batch, W1, b1, W2, b2):
    del batch
    n = weights.shape[0]
    grid = (n // BLOCK_N,)

    posT = positions.T
    W2pT = W2[:, :POS_DIM].T
    W2w = W2[:, POS_DIM:]
    b1r = b1.reshape(1, HIDDEN)
    b2pT = b2[:POS_DIM].reshape(POS_DIM, 1)
    b2w = b2[POS_DIM:].reshape(1, FEAT_DIM)

    row_block = lambda i: (i, 0)
    col_block = lambda i: (0, i)
    rep = lambda i: (0, 0)
    out_posT, out_w = pl.pallas_call(
        _mlp_block_kernel,
        grid=grid,
        in_specs=[
            pl.BlockSpec((POS_DIM, BLOCK_N), col_block),
            pl.BlockSpec((BLOCK_N, FEAT_DIM), row_block),
            pl.BlockSpec((FEAT_DIM, HIDDEN), rep),
            pl.BlockSpec((1, HIDDEN), rep),
            pl.BlockSpec((POS_DIM, HIDDEN), rep),
            pl.BlockSpec((POS_DIM, 1), rep),
            pl.BlockSpec((HIDDEN, FEAT_DIM), rep),
            pl.BlockSpec((1, FEAT_DIM), rep),
        ],
        out_specs=[
            pl.BlockSpec((POS_DIM, BLOCK_N), col_block),
            pl.BlockSpec((BLOCK_N, FEAT_DIM), row_block),
        ],
        out_shape=[
            jax.ShapeDtypeStruct((POS_DIM, n), jnp.float32),
            jax.ShapeDtypeStruct((n, FEAT_DIM), jnp.float32),
        ],
        compiler_params=pltpu.CompilerParams(
            dimension_semantics=("parallel",),
        ),
    )(posT, weights, W1, b1r, W2pT, b2pT, W2w, b2w)
    return out_posT.T, out_w


# speedup vs baseline: 1.1452x; 1.0312x over previous
import jax
import jax.numpy as jnp
from jax import lax
from jax.experimental import pallas as pl
from jax.experimental.pallas import tpu as pltpu

POS_DIM = 3
FEAT_DIM = 128
HIDDEN = 256
BLOCK_N = 8192


def _mlp_block_kernel(post_ref, w_ref, w1_ref, b1_ref, w2_ref, b2pt_ref,
                      b2w_ref, out_post_ref, out_w_ref):
    w = w_ref[...]
    h = jnp.maximum(
        jnp.dot(w, w1_ref[...], preferred_element_type=jnp.float32)
        + b1_ref[...], 0.0)
    w2 = w2_ref[...]
    dpt = lax.dot_general(w2[:, :POS_DIM], h, (((0,), (1,)), ((), ())),
                          preferred_element_type=jnp.float32)
    dw = jnp.dot(h, w2[:, POS_DIM:], preferred_element_type=jnp.float32)
    out_post_ref[...] = post_ref[...] + dpt + b2pt_ref[...]
    out_w_ref[...] = w + dw + b2w_ref[...]


def kernel(positions, weights, batch, W1, b1, W2, b2):
    del batch
    n = weights.shape[0]
    grid = (n // BLOCK_N,)

    posT = positions.T
    b1r = b1.reshape(1, HIDDEN)
    b2pT = b2[:POS_DIM].reshape(POS_DIM, 1)
    b2w = b2[POS_DIM:].reshape(1, FEAT_DIM)

    row_block = lambda i: (i, 0)
    col_block = lambda i: (0, i)
    rep = lambda i: (0, 0)
    out_posT, out_w = pl.pallas_call(
        _mlp_block_kernel,
        grid=grid,
        in_specs=[
            pl.BlockSpec((POS_DIM, BLOCK_N), col_block),
            pl.BlockSpec((BLOCK_N, FEAT_DIM), row_block),
            pl.BlockSpec((FEAT_DIM, HIDDEN), rep),
            pl.BlockSpec((1, HIDDEN), rep),
            pl.BlockSpec((HIDDEN, POS_DIM + FEAT_DIM), rep),
            pl.BlockSpec((POS_DIM, 1), rep),
            pl.BlockSpec((1, FEAT_DIM), rep),
        ],
        out_specs=[
            pl.BlockSpec((POS_DIM, BLOCK_N), col_block),
            pl.BlockSpec((BLOCK_N, FEAT_DIM), row_block),
        ],
        out_shape=[
            jax.ShapeDtypeStruct((POS_DIM, n), jnp.float32),
            jax.ShapeDtypeStruct((n, FEAT_DIM), jnp.float32),
        ],
        compiler_params=pltpu.CompilerParams(
            dimension_semantics=("parallel",),
        ),
    )(posT, weights, W1, b1r, W2, b2pT, b2w)
    return out_posT.T, out_w


# speedup vs baseline: 1.2258x; 1.0704x over previous
import jax
import jax.numpy as jnp
from jax import lax
from jax.experimental import pallas as pl
from jax.experimental.pallas import tpu as pltpu

POS_DIM = 3
FEAT_DIM = 128
HIDDEN = 256
BLOCK_N = 8192


def _mlp_block_kernel(post_ref, w_ref, w1_ref, b1_ref, w2_ref, b2_ref,
                      out_post_ref, out_w_ref):
    w = w_ref[...]
    h = jnp.maximum(
        jnp.dot(w, w1_ref[...], preferred_element_type=jnp.float32)
        + b1_ref[...], 0.0)
    w2 = w2_ref[...]
    dpt = lax.dot_general(w2[:, :POS_DIM], h, (((0,), (1,)), ((), ())),
                          preferred_element_type=jnp.float32)
    dw = jnp.dot(h, w2[:, POS_DIM:], preferred_element_type=jnp.float32)
    b2 = b2_ref[...]
    out_post_ref[...] = post_ref[...] + dpt + jnp.transpose(b2[:, :POS_DIM])
    out_w_ref[...] = w + dw + b2[:, POS_DIM:]


def kernel(positions, weights, batch, W1, b1, W2, b2):
    del batch
    n = weights.shape[0]
    grid = (n // BLOCK_N,)

    posT = positions.T
    b1r = b1.reshape(1, HIDDEN)
    b2r = b2.reshape(1, POS_DIM + FEAT_DIM)

    row_block = lambda i: (i, 0)
    col_block = lambda i: (0, i)
    rep = lambda i: (0, 0)
    out_posT, out_w = pl.pallas_call(
        _mlp_block_kernel,
        grid=grid,
        in_specs=[
            pl.BlockSpec((POS_DIM, BLOCK_N), col_block),
            pl.BlockSpec((BLOCK_N, FEAT_DIM), row_block),
            pl.BlockSpec((FEAT_DIM, HIDDEN), rep),
            pl.BlockSpec((1, HIDDEN), rep),
            pl.BlockSpec((HIDDEN, POS_DIM + FEAT_DIM), rep),
            pl.BlockSpec((1, POS_DIM + FEAT_DIM), rep),
        ],
        out_specs=[
            pl.BlockSpec((POS_DIM, BLOCK_N), col_block),
            pl.BlockSpec((BLOCK_N, FEAT_DIM), row_block),
        ],
        out_shape=[
            jax.ShapeDtypeStruct((POS_DIM, n), jnp.float32),
            jax.ShapeDtypeStruct((n, FEAT_DIM), jnp.float32),
        ],
        compiler_params=pltpu.CompilerParams(
            dimension_semantics=("parallel",),
        ),
    )(posT, weights, W1, b1r, W2, b2r)
    return out_posT.T, out_w
